# Initial kernel scaffold; baseline (speedup 1.0000x reference)
import jax
import jax.numpy as jnp
from jax.experimental import pallas as pl


def kernel(x, edge_index, W0, b0, W1, b1, prelu_a, gamma, beta):
    raise NotImplementedError("write your pallas kernel here")


# speedup vs baseline: 1.5573x; 1.5573x over previous
import functools

import jax
import jax.numpy as jnp
from jax import lax
from jax.experimental import pallas as pl
from jax.experimental.pallas import tpu as pltpu
from jax.experimental.pallas import tpu_sc as plsc

N = 4096
E = 65536
IN_CH = 256
HIDDEN = 64
OUT_CH = 256
BLK = 512
NB = N // BLK

_NTILES = 32
_EPT = E // _NTILES
_ROWS_PT = _EPT // 128


@functools.lru_cache(maxsize=None)
def _sc_kernels():
    mesh = plsc.VectorSubcoreMesh(core_axis_name="c", subcore_axis_name="s")

    @functools.partial(
        pl.kernel, mesh=mesh,
        out_type=jax.ShapeDtypeStruct((2, N, 16), jnp.float32),
        compiler_params=pltpu.CompilerParams(use_tc_tiling_on_sc=False),
        scratch_types=[
            pltpu.VMEM((_ROWS_PT, 128), jnp.int32),
            pltpu.VMEM((128, 16), jnp.float32),
            pltpu.VMEM_SHARED((N, 16), jnp.float32),
        ],
    )
    def sc_deg(dst2d, zer16, ones16, out, idx_v, ones_v, hist_sh):
        c = lax.axis_index("c")
        s = lax.axis_index("s")
        wid = s * 2 + c

        @pl.when(s == 0)
        def _():
            pltpu.sync_copy(zer16, hist_sh)

        plsc.subcore_barrier()
        pltpu.sync_copy(dst2d.at[pl.ds(wid * _ROWS_PT, _ROWS_PT)], idx_v)
        pltpu.sync_copy(ones16, ones_v)
        for j in range(_ROWS_PT):
            pltpu.sync_copy(ones_v, hist_sh.at[idx_v.at[j]], add=True)
        plsc.subcore_barrier()
        pltpu.sync_copy(hist_sh.at[pl.ds(s * (N // 16), N // 16)],
                        out.at[c, pl.ds(s * (N // 16), N // 16)])

    @functools.partial(
        pl.kernel, mesh=mesh,
        out_type=jax.ShapeDtypeStruct((2, N, HIDDEN), jnp.float32),
        compiler_params=pltpu.CompilerParams(use_tc_tiling_on_sc=False),
        scratch_types=[
            pltpu.VMEM((_ROWS_PT, 128), jnp.int32),
            pltpu.VMEM((_ROWS_PT, 128), jnp.int32),
            pltpu.VMEM((128, HIDDEN), jnp.float32),
            pltpu.VMEM_SHARED((N, HIDDEN), jnp.float32),
            pltpu.SemaphoreType.DMA,
        ],
    )
    def sc_agg(src2d, dst2d, y0, zer64, out, isrc, idst, rows_v, agg_sh, sem):
        c = lax.axis_index("c")
        s = lax.axis_index("s")
        wid = s * 2 + c

        @pl.when(s == 0)
        def _():
            pltpu.sync_copy(zer64, agg_sh)

        plsc.subcore_barrier()
        pltpu.sync_copy(src2d.at[pl.ds(wid * _ROWS_PT, _ROWS_PT)], isrc)
        pltpu.sync_copy(dst2d.at[pl.ds(wid * _ROWS_PT, _ROWS_PT)], idst)
        for j in range(_ROWS_PT):
            pltpu.async_copy(y0.at[isrc.at[j]], rows_v, sem).wait()
            pltpu.sync_copy(rows_v, agg_sh.at[idst.at[j]], add=True)
        plsc.subcore_barrier()
        pltpu.sync_copy(agg_sh.at[pl.ds(s * (N // 16), N // 16)],
                        out.at[c, pl.ds(s * (N // 16), N // 16)])

    @functools.partial(
        pl.kernel, mesh=mesh,
        out_type=jax.ShapeDtypeStruct((8,), jnp.float32),
        scratch_types=[
            pltpu.VMEM((_ROWS_PT, 128), jnp.int32),
            pltpu.VMEM((128,), jnp.float32),
        ],
    )
    def sc_orig(flat2d, ones1d, adj_ref, dummy, idx_v, ones_v):
        c = lax.axis_index("c")
        s = lax.axis_index("s")
        wid = s * 2 + c
        pltpu.sync_copy(flat2d.at[pl.ds(wid * _ROWS_PT, _ROWS_PT)], idx_v)
        pltpu.sync_copy(ones1d, ones_v)
        for j in range(_ROWS_PT):
            pltpu.sync_copy(ones_v, adj_ref.at[idx_v.at[j]])

        @pl.when(jnp.logical_and(c == 0, s == 0))
        def _():
            pltpu.sync_copy(ones_v.at[pl.ds(0, 8)], dummy)

    return sc_deg, sc_agg, sc_orig


def _prep_body(x_ref, w0_ref, degp_ref, xw_ref, y0_ref):
    xw = jnp.dot(x_ref[...].astype(jnp.bfloat16),
                 w0_ref[...].astype(jnp.bfloat16),
                 preferred_element_type=jnp.float32)
    deg = degp_ref[0, :, 0:1] + degp_ref[1, :, 0:1] + 1.0
    dis = 1.0 / jnp.sqrt(deg)
    xw_ref[...] = xw
    y0_ref[...] = (dis * xw).astype(jnp.bfloat16).astype(jnp.float32)


def _h_body(aggp_ref, y0_ref, degp_ref, b0_ref, h_ref):
    deg = degp_ref[0, :, 0:1] + degp_ref[1, :, 0:1] + 1.0
    dis = 1.0 / jnp.sqrt(deg)
    agg = aggp_ref[0] + aggp_ref[1] + y0_ref[...]
    h_ref[...] = dis * agg + b0_ref[...]


def _logits_body(ha_ref, hb_ref, out_ref, mx_ref):
    i = pl.program_id(0)
    j = pl.program_id(1)
    t = lax.dot_general(ha_ref[...].astype(jnp.bfloat16),
                        hb_ref[...].astype(jnp.bfloat16),
                        (((1,), (1,)), ((), ())),
                        preferred_element_type=jnp.float32)
    out_ref[...] = t
    prev = jnp.where((i == 0) & (j == 0),
                     jnp.full((1, 1), -jnp.inf, jnp.float32), mx_ref[...])
    mx_ref[...] = jnp.maximum(prev, jnp.max(t).reshape(1, 1))


def _sample_body(ha_ref, hb_ref, orig_ref, u_ref, m_ref, samp_ref, rs_ref):
    i = pl.program_id(0)
    j = pl.program_id(1)
    m = m_ref[0, 0]
    logits = lax.dot_general(ha_ref[...].astype(jnp.bfloat16),
                             hb_ref[...].astype(jnp.bfloat16),
                             (((1,), (1,)), ((), ())),
                             preferred_element_type=jnp.float32)
    ep = 0.8 * (logits / m) + 0.2 * orig_ref[...]
    ep = jnp.where(ep < 0.0, 0.0, ep)
    p = jnp.clip(ep, 1e-6, 1.0 - 1e-6)
    lg = jnp.log(p) - jnp.log1p(-p)
    u = u_ref[...]
    t = lg + jnp.log(u) - jnp.log1p(-u)
    soft = jnp.where(t >= 0.0,
                     1.0 / (1.0 + jnp.exp(-t)),
                     jnp.exp(t) / (1.0 + jnp.exp(t)))
    r = jnp.round(soft)
    rows = lax.broadcasted_iota(jnp.int32, (BLK, BLK), 0)
    cols = lax.broadcasted_iota(jnp.int32, (BLK, BLK), 1)
    tri = jnp.where(rows < cols, r, 0.0)
    sym = tri + tri.T + jnp.where(rows == cols, 1.0, 0.0)
    out = jnp.where(i < j, r, jnp.where(i > j, r.T, sym))
    samp_ref[...] = out
    prev = jnp.where(j == 0, jnp.zeros((BLK, 1), jnp.float32), rs_ref[...])
    rs_ref[...] = prev + jnp.sum(out, axis=1, keepdims=True)


def _h1_body(a_ref, xwk_ref, csk_ref, xwi_ref, csi_ref, b0_ref, pa_ref,
             h1_ref, z_ref, sm_ref, sq_ref, acc_ref):
    i = pl.program_id(0)
    k = pl.program_id(1)
    dis_k = lax.rsqrt(csk_ref[...] + 1.0)
    y1_k = xwk_ref[...] * dis_k

    @pl.when(k == 0)
    def _():
        acc_ref[...] = jnp.zeros_like(acc_ref)

    acc_ref[...] += jnp.dot(a_ref[...], y1_k,
                            preferred_element_type=jnp.float32)

    @pl.when(k == NB - 1)
    def _():
        dis_i = lax.rsqrt(csi_ref[...] + 1.0)
        y1_i = xwi_ref[...] * dis_i
        h1 = dis_i * (acc_ref[...] + y1_i) + b0_ref[...]
        h1_ref[...] = h1
        a = pa_ref[0, 0]
        z_ref[...] = jnp.where(h1 >= 0.0, h1, a * h1)
        pv_s = jnp.where(i == 0, jnp.zeros((1, HIDDEN), jnp.float32),
                         sm_ref[...])
        pv_q = jnp.where(i == 0, jnp.zeros((1, HIDDEN), jnp.float32),
                         sq_ref[...])
        sm_ref[...] = pv_s + jnp.sum(h1, axis=0, keepdims=True)
        sq_ref[...] = pv_q + jnp.sum(h1 * h1, axis=0, keepdims=True)


def _out_body(a_ref, h1k_ref, csk_ref, h1i_ref, csi_ref, sm_ref, sq_ref,
              gam_ref, bet_ref, w1_ref, b1_ref, out_ref, acc_ref, gself_ref):
    k = pl.program_id(1)
    mean = sm_ref[...] / N
    var = sq_ref[...] / N - mean * mean
    inv = lax.rsqrt(var + 1e-5)

    def bn_g(h1, cs):
        hbn = (h1 - mean) * inv * gam_ref[...] + bet_ref[...]
        h2 = jnp.maximum(hbn, 0.0)
        return lax.rsqrt(cs + 1.0) * h2

    g_k = bn_g(h1k_ref[...], csk_ref[...])

    @pl.when(k == 0)
    def _():
        acc_ref[...] = jnp.zeros_like(acc_ref)

    acc_ref[...] += jnp.dot(a_ref[...], g_k,
                            preferred_element_type=jnp.float32)

    @pl.when(k == NB - 1)
    def _():
        g_i = bn_g(h1i_ref[...], csi_ref[...])
        s = acc_ref[...] + g_i
        dis_i = lax.rsqrt(csi_ref[...] + 1.0)
        out_ref[...] = dis_i * jnp.dot(s, w1_ref[...],
                                       preferred_element_type=jnp.float32) \
            + b1_ref[...]


def _vspec(shape, imap):
    return pl.BlockSpec(shape, imap)


_SMEM_SPEC = pl.BlockSpec(memory_space=pltpu.SMEM)


def kernel(x, edge_index, W0, b0, W1, b1, prelu_a, gamma, beta):
    src = edge_index[0].astype(jnp.int32)
    dst = edge_index[1].astype(jnp.int32)
    src2d = src.reshape(E // 128, 128)
    dst2d = dst.reshape(E // 128, 128)
    flat2d = (src * N + dst).reshape(E // 128, 128)
    u = jax.random.uniform(jax.random.key(42), (N, N),
                           minval=1e-6, maxval=1.0 - 1e-6)
    zer16 = jnp.zeros((N, 16), jnp.float32)
    ones16 = jnp.ones((128, 16), jnp.float32)
    zer64 = jnp.zeros((N, HIDDEN), jnp.float32)
    ones1d = jnp.ones((128,), jnp.float32)
    b0r = b0.reshape(1, HIDDEN)
    b1r = b1.reshape(1, OUT_CH)
    gam = gamma.reshape(1, HIDDEN)
    bet = beta.reshape(1, HIDDEN)
    pa = prelu_a.reshape(1, 1)

    sc_deg, sc_agg, sc_orig = _sc_kernels()

    degp = sc_deg(dst2d, zer16, ones16)

    xw0, y0 = pl.pallas_call(
        _prep_body,
        grid=(NB,),
        in_specs=[
            _vspec((BLK, IN_CH), lambda i: (i, 0)),
            _vspec((IN_CH, HIDDEN), lambda i: (0, 0)),
            _vspec((2, BLK, 16), lambda i: (0, i, 0)),
        ],
        out_specs=[
            _vspec((BLK, HIDDEN), lambda i: (i, 0)),
            _vspec((BLK, HIDDEN), lambda i: (i, 0)),
        ],
        out_shape=[
            jax.ShapeDtypeStruct((N, HIDDEN), jnp.float32),
            jax.ShapeDtypeStruct((N, HIDDEN), jnp.float32),
        ],
    )(x, W0, degp)

    aggp = sc_agg(src2d, dst2d, y0, zer64)

    h = pl.pallas_call(
        _h_body,
        grid=(NB,),
        in_specs=[
            _vspec((2, BLK, HIDDEN), lambda i: (0, i, 0)),
            _vspec((BLK, HIDDEN), lambda i: (i, 0)),
            _vspec((2, BLK, 16), lambda i: (0, i, 0)),
            _vspec((1, HIDDEN), lambda i: (0, 0)),
        ],
        out_specs=_vspec((BLK, HIDDEN), lambda i: (i, 0)),
        out_shape=jax.ShapeDtypeStruct((N, HIDDEN), jnp.float32),
    )(aggp, y0, degp, b0r)

    adj_logits, mx = pl.pallas_call(
        _logits_body,
        grid=(NB, NB),
        in_specs=[
            _vspec((BLK, HIDDEN), lambda i, j: (i, 0)),
            _vspec((BLK, HIDDEN), lambda i, j: (j, 0)),
        ],
        out_specs=[
            _vspec((BLK, BLK), lambda i, j: (i, j)),
            _vspec((1, 1), lambda i, j: (0, 0)),
        ],
        out_shape=[
            jax.ShapeDtypeStruct((N, N), jnp.float32),
            jax.ShapeDtypeStruct((1, 1), jnp.float32),
        ],
    )(h, h)

    adjr = jax.new_ref(jnp.zeros((N * N,), jnp.float32))
    sc_orig(flat2d, ones1d, adjr)
    adj_orig = adjr[...].reshape(N, N)

    adj_sampled, rowsum = pl.pallas_call(
        _sample_body,
        grid=(NB, NB),
        in_specs=[
            _vspec((BLK, HIDDEN), lambda i, j: (jnp.minimum(i, j), 0)),
            _vspec((BLK, HIDDEN), lambda i, j: (jnp.maximum(i, j), 0)),
            _vspec((BLK, BLK),
                   lambda i, j: (jnp.minimum(i, j), jnp.maximum(i, j))),
            _vspec((BLK, BLK),
                   lambda i, j: (jnp.minimum(i, j), jnp.maximum(i, j))),
            _SMEM_SPEC,
        ],
        out_specs=[
            _vspec((BLK, BLK), lambda i, j: (i, j)),
            _vspec((BLK, 1), lambda i, j: (i, 0)),
        ],
        out_shape=[
            jax.ShapeDtypeStruct((N, N), jnp.float32),
            jax.ShapeDtypeStruct((N, 1), jnp.float32),
        ],
    )(h, h, adj_orig, u, mx)

    h1, z, sm, sq = pl.pallas_call(
        _h1_body,
        grid=(NB, NB),
        in_specs=[
            _vspec((BLK, BLK), lambda i, k: (i, k)),
            _vspec((BLK, HIDDEN), lambda i, k: (k, 0)),
            _vspec((BLK, 1), lambda i, k: (k, 0)),
            _vspec((BLK, HIDDEN), lambda i, k: (i, 0)),
            _vspec((BLK, 1), lambda i, k: (i, 0)),
            _vspec((1, HIDDEN), lambda i, k: (0, 0)),
            _SMEM_SPEC,
        ],
        out_specs=[
            _vspec((BLK, HIDDEN), lambda i, k: (i, 0)),
            _vspec((BLK, HIDDEN), lambda i, k: (i, 0)),
            _vspec((1, HIDDEN), lambda i, k: (0, 0)),
            _vspec((1, HIDDEN), lambda i, k: (0, 0)),
        ],
        out_shape=[
            jax.ShapeDtypeStruct((N, HIDDEN), jnp.float32),
            jax.ShapeDtypeStruct((N, HIDDEN), jnp.float32),
            jax.ShapeDtypeStruct((1, HIDDEN), jnp.float32),
            jax.ShapeDtypeStruct((1, HIDDEN), jnp.float32),
        ],
        scratch_shapes=[pltpu.VMEM((BLK, HIDDEN), jnp.float32)],
    )(adj_sampled, xw0, rowsum, xw0, rowsum, b0r, pa)

    out = pl.pallas_call(
        _out_body,
        grid=(NB, NB),
        in_specs=[
            _vspec((BLK, BLK), lambda i, k: (i, k)),
            _vspec((BLK, HIDDEN), lambda i, k: (k, 0)),
            _vspec((BLK, 1), lambda i, k: (k, 0)),
            _vspec((BLK, HIDDEN), lambda i, k: (i, 0)),
            _vspec((BLK, 1), lambda i, k: (i, 0)),
            _vspec((1, HIDDEN), lambda i, k: (0, 0)),
            _vspec((1, HIDDEN), lambda i, k: (0, 0)),
            _vspec((1, HIDDEN), lambda i, k: (0, 0)),
            _vspec((1, HIDDEN), lambda i, k: (0, 0)),
            _vspec((HIDDEN, OUT_CH), lambda i, k: (0, 0)),
            _vspec((1, OUT_CH), lambda i, k: (0, 0)),
        ],
        out_specs=_vspec((BLK, OUT_CH), lambda i, k: (i, 0)),
        out_shape=jax.ShapeDtypeStruct((N, OUT_CH), jnp.float32),
        scratch_shapes=[pltpu.VMEM((BLK, HIDDEN), jnp.float32),
                        pltpu.VMEM((BLK, HIDDEN), jnp.float32)],
    )(adj_sampled, h1, rowsum, h1, rowsum, sm, sq, gam, bet, W1, b1r)

    return (out, z, adj_sampled, adj_logits, adj_orig)


# speedup vs baseline: 1.5780x; 1.0133x over previous
import functools

import jax
import jax.numpy as jnp
from jax import lax
from jax.experimental import pallas as pl
from jax.experimental.pallas import tpu as pltpu
from jax.experimental.pallas import tpu_sc as plsc

N = 4096
E = 65536
IN_CH = 256
HIDDEN = 64
OUT_CH = 256
BLK = 512
NB = N // BLK

_NTILES = 32
_EPT = E // _NTILES
_ROWS_PT = _EPT // 128


@functools.lru_cache(maxsize=None)
def _sc_kernels():
    mesh = plsc.VectorSubcoreMesh(core_axis_name="c", subcore_axis_name="s")

    @functools.partial(
        pl.kernel, mesh=mesh,
        out_type=jax.ShapeDtypeStruct((2, N, 16), jnp.float32),
        compiler_params=pltpu.CompilerParams(use_tc_tiling_on_sc=False),
        scratch_types=[
            pltpu.VMEM((_ROWS_PT, 128), jnp.int32),
            pltpu.VMEM((128, 16), jnp.float32),
            pltpu.VMEM_SHARED((N, 16), jnp.float32),
        ],
    )
    def sc_deg(dst2d, zer16, ones16, out, idx_v, ones_v, hist_sh):
        c = lax.axis_index("c")
        s = lax.axis_index("s")
        wid = s * 2 + c

        @pl.when(s == 0)
        def _():
            pltpu.sync_copy(zer16, hist_sh)

        plsc.subcore_barrier()
        pltpu.sync_copy(dst2d.at[pl.ds(wid * _ROWS_PT, _ROWS_PT)], idx_v)
        pltpu.sync_copy(ones16, ones_v)
        for j in range(_ROWS_PT):
            pltpu.sync_copy(ones_v, hist_sh.at[idx_v.at[j]], add=True)
        plsc.subcore_barrier()
        pltpu.sync_copy(hist_sh.at[pl.ds(s * (N // 16), N // 16)],
                        out.at[c, pl.ds(s * (N // 16), N // 16)])

    @functools.partial(
        pl.kernel, mesh=mesh,
        out_type=jax.ShapeDtypeStruct((2, N, HIDDEN), jnp.float32),
        compiler_params=pltpu.CompilerParams(use_tc_tiling_on_sc=False),
        scratch_types=[
            pltpu.VMEM((_ROWS_PT, 128), jnp.int32),
            pltpu.VMEM((_ROWS_PT, 128), jnp.int32),
            pltpu.VMEM((128, HIDDEN), jnp.float32),
            pltpu.VMEM((128, HIDDEN), jnp.float32),
            pltpu.VMEM_SHARED((N, HIDDEN), jnp.float32),
            pltpu.SemaphoreType.DMA,
            pltpu.SemaphoreType.DMA,
        ],
    )
    def sc_agg(src2d, dst2d, y0, zer64, out,
               isrc, idst, rows_v, rows_v2, agg_sh, sem, sem2):
        c = lax.axis_index("c")
        s = lax.axis_index("s")
        wid = s * 2 + c

        @pl.when(s == 0)
        def _():
            pltpu.sync_copy(zer64, agg_sh)

        plsc.subcore_barrier()
        pltpu.sync_copy(src2d.at[pl.ds(wid * _ROWS_PT, _ROWS_PT)], isrc)
        pltpu.sync_copy(dst2d.at[pl.ds(wid * _ROWS_PT, _ROWS_PT)], idst)
        bufs = (rows_v, rows_v2)
        sems = (sem, sem2)
        descs = [None] * _ROWS_PT
        descs[0] = pltpu.async_copy(y0.at[isrc.at[0]], bufs[0], sems[0])
        for j in range(_ROWS_PT):
            descs[j].wait()
            if j + 1 < _ROWS_PT:
                descs[j + 1] = pltpu.async_copy(
                    y0.at[isrc.at[j + 1]], bufs[(j + 1) % 2], sems[(j + 1) % 2])
            pltpu.sync_copy(bufs[j % 2], agg_sh.at[idst.at[j]], add=True)
        plsc.subcore_barrier()
        pltpu.sync_copy(agg_sh.at[pl.ds(s * (N // 16), N // 16)],
                        out.at[c, pl.ds(s * (N // 16), N // 16)])

    @functools.partial(
        pl.kernel, mesh=mesh,
        out_type=jax.ShapeDtypeStruct((8,), jnp.float32),
        scratch_types=[
            pltpu.VMEM((_ROWS_PT, 128), jnp.int32),
            pltpu.VMEM((128,), jnp.float32),
        ],
    )
    def sc_orig(flat2d, ones1d, adj_ref, dummy, idx_v, ones_v):
        c = lax.axis_index("c")
        s = lax.axis_index("s")
        wid = s * 2 + c
        pltpu.sync_copy(flat2d.at[pl.ds(wid * _ROWS_PT, _ROWS_PT)], idx_v)
        pltpu.sync_copy(ones1d, ones_v)
        for j in range(_ROWS_PT):
            pltpu.sync_copy(ones_v, adj_ref.at[idx_v.at[j]])

        @pl.when(jnp.logical_and(c == 0, s == 0))
        def _():
            pltpu.sync_copy(ones_v.at[pl.ds(0, 8)], dummy)

    return sc_deg, sc_agg, sc_orig


def _prep_body(x_ref, w0_ref, degp_ref, xw_ref, y0_ref):
    xw = jnp.dot(x_ref[...].astype(jnp.bfloat16),
                 w0_ref[...].astype(jnp.bfloat16),
                 preferred_element_type=jnp.float32)
    deg = degp_ref[0, :, 0:1] + degp_ref[1, :, 0:1] + 1.0
    dis = 1.0 / jnp.sqrt(deg)
    xw_ref[...] = xw
    y0_ref[...] = (dis * xw).astype(jnp.bfloat16).astype(jnp.float32)


def _h_body(aggp_ref, y0_ref, degp_ref, b0_ref, h_ref):
    deg = degp_ref[0, :, 0:1] + degp_ref[1, :, 0:1] + 1.0
    dis = 1.0 / jnp.sqrt(deg)
    agg = aggp_ref[0] + aggp_ref[1] + y0_ref[...]
    h_ref[...] = dis * agg + b0_ref[...]


def _logits_body(ha_ref, hb_ref, out_ref, mx_ref):
    i = pl.program_id(0)
    j = pl.program_id(1)
    t = lax.dot_general(ha_ref[...].astype(jnp.bfloat16),
                        hb_ref[...].astype(jnp.bfloat16),
                        (((1,), (1,)), ((), ())),
                        preferred_element_type=jnp.float32)
    out_ref[...] = t
    prev = jnp.where((i == 0) & (j == 0),
                     jnp.full((1, 1), -jnp.inf, jnp.float32), mx_ref[...])
    mx_ref[...] = jnp.maximum(prev, jnp.max(t).reshape(1, 1))


def _sample_body(ha_ref, hb_ref, orig_ref, u_ref, m_ref,
                 samp_ref, samp16_ref, rs_ref):
    i = pl.program_id(0)
    j = pl.program_id(1)
    m = m_ref[0, 0]
    logits = lax.dot_general(ha_ref[...].astype(jnp.bfloat16),
                             hb_ref[...].astype(jnp.bfloat16),
                             (((1,), (1,)), ((), ())),
                             preferred_element_type=jnp.float32)
    ep = 0.8 * (logits / m) + 0.2 * orig_ref[...]
    ep = jnp.where(ep < 0.0, 0.0, ep)
    p = jnp.clip(ep, 1e-6, 1.0 - 1e-6)
    lg = jnp.log(p) - jnp.log1p(-p)
    u = u_ref[...]
    t = lg + jnp.log(u) - jnp.log1p(-u)
    soft = jnp.where(t >= 0.0,
                     1.0 / (1.0 + jnp.exp(-t)),
                     jnp.exp(t) / (1.0 + jnp.exp(t)))
    r = jnp.round(soft)
    rows = lax.broadcasted_iota(jnp.int32, (BLK, BLK), 0)
    cols = lax.broadcasted_iota(jnp.int32, (BLK, BLK), 1)
    tri = jnp.where(rows < cols, r, 0.0)
    sym = tri + tri.T + jnp.where(rows == cols, 1.0, 0.0)
    out = jnp.where(i < j, r, jnp.where(i > j, r.T, sym))
    samp_ref[...] = out
    samp16_ref[...] = out.astype(jnp.bfloat16)
    prev = jnp.where(j == 0, jnp.zeros((BLK, 1), jnp.float32), rs_ref[...])
    rs_ref[...] = prev + jnp.sum(out, axis=1, keepdims=True)


def _h1_body(a_ref, xwk_ref, csk_ref, xwi_ref, csi_ref, b0_ref, pa_ref,
             h1_ref, z_ref, sm_ref, sq_ref, acc_ref):
    i = pl.program_id(0)
    k = pl.program_id(1)
    dis_k = lax.rsqrt(csk_ref[...] + 1.0)
    y1_k = xwk_ref[...] * dis_k

    @pl.when(k == 0)
    def _():
        acc_ref[...] = jnp.zeros_like(acc_ref)

    acc_ref[...] += jnp.dot(a_ref[...], y1_k.astype(jnp.bfloat16),
                            preferred_element_type=jnp.float32)

    @pl.when(k == NB - 1)
    def _():
        dis_i = lax.rsqrt(csi_ref[...] + 1.0)
        y1_i = xwi_ref[...] * dis_i
        h1 = dis_i * (acc_ref[...] + y1_i) + b0_ref[...]
        h1_ref[...] = h1
        a = pa_ref[0, 0]
        z_ref[...] = jnp.where(h1 >= 0.0, h1, a * h1)
        pv_s = jnp.where(i == 0, jnp.zeros((1, HIDDEN), jnp.float32),
                         sm_ref[...])
        pv_q = jnp.where(i == 0, jnp.zeros((1, HIDDEN), jnp.float32),
                         sq_ref[...])
        sm_ref[...] = pv_s + jnp.sum(h1, axis=0, keepdims=True)
        sq_ref[...] = pv_q + jnp.sum(h1 * h1, axis=0, keepdims=True)


def _out_body(a_ref, h1k_ref, csk_ref, h1i_ref, csi_ref, sm_ref, sq_ref,
              gam_ref, bet_ref, w1_ref, b1_ref, out_ref, acc_ref, gself_ref):
    k = pl.program_id(1)
    mean = sm_ref[...] / N
    var = sq_ref[...] / N - mean * mean
    inv = lax.rsqrt(var + 1e-5)

    def bn_g(h1, cs):
        hbn = (h1 - mean) * inv * gam_ref[...] + bet_ref[...]
        h2 = jnp.maximum(hbn, 0.0)
        return lax.rsqrt(cs + 1.0) * h2

    g_k = bn_g(h1k_ref[...], csk_ref[...])

    @pl.when(k == 0)
    def _():
        acc_ref[...] = jnp.zeros_like(acc_ref)

    acc_ref[...] += jnp.dot(a_ref[...], g_k.astype(jnp.bfloat16),
                            preferred_element_type=jnp.float32)

    @pl.when(k == NB - 1)
    def _():
        g_i = bn_g(h1i_ref[...], csi_ref[...])
        s = acc_ref[...] + g_i
        dis_i = lax.rsqrt(csi_ref[...] + 1.0)
        out_ref[...] = dis_i * jnp.dot(s, w1_ref[...],
                                       preferred_element_type=jnp.float32) \
            + b1_ref[...]


def _vspec(shape, imap):
    return pl.BlockSpec(shape, imap)


_SMEM_SPEC = pl.BlockSpec(memory_space=pltpu.SMEM)


def kernel(x, edge_index, W0, b0, W1, b1, prelu_a, gamma, beta):
    src = edge_index[0].astype(jnp.int32)
    dst = edge_index[1].astype(jnp.int32)
    src2d = src.reshape(E // 128, 128)
    dst2d = dst.reshape(E // 128, 128)
    flat2d = (src * N + dst).reshape(E // 128, 128)
    u = jax.random.uniform(jax.random.key(42), (N, N),
                           minval=1e-6, maxval=1.0 - 1e-6)
    zer16 = jnp.zeros((N, 16), jnp.float32)
    ones16 = jnp.ones((128, 16), jnp.float32)
    zer64 = jnp.zeros((N, HIDDEN), jnp.float32)
    ones1d = jnp.ones((128,), jnp.float32)
    b0r = b0.reshape(1, HIDDEN)
    b1r = b1.reshape(1, OUT_CH)
    gam = gamma.reshape(1, HIDDEN)
    bet = beta.reshape(1, HIDDEN)
    pa = prelu_a.reshape(1, 1)

    sc_deg, sc_agg, sc_orig = _sc_kernels()

    degp = sc_deg(dst2d, zer16, ones16)

    xw0, y0 = pl.pallas_call(
        _prep_body,
        grid=(NB,),
        in_specs=[
            _vspec((BLK, IN_CH), lambda i: (i, 0)),
            _vspec((IN_CH, HIDDEN), lambda i: (0, 0)),
            _vspec((2, BLK, 16), lambda i: (0, i, 0)),
        ],
        out_specs=[
            _vspec((BLK, HIDDEN), lambda i: (i, 0)),
            _vspec((BLK, HIDDEN), lambda i: (i, 0)),
        ],
        out_shape=[
            jax.ShapeDtypeStruct((N, HIDDEN), jnp.float32),
            jax.ShapeDtypeStruct((N, HIDDEN), jnp.float32),
        ],
    )(x, W0, degp)

    aggp = sc_agg(src2d, dst2d, y0, zer64)

    h = pl.pallas_call(
        _h_body,
        grid=(NB,),
        in_specs=[
            _vspec((2, BLK, HIDDEN), lambda i: (0, i, 0)),
            _vspec((BLK, HIDDEN), lambda i: (i, 0)),
            _vspec((2, BLK, 16), lambda i: (0, i, 0)),
            _vspec((1, HIDDEN), lambda i: (0, 0)),
        ],
        out_specs=_vspec((BLK, HIDDEN), lambda i: (i, 0)),
        out_shape=jax.ShapeDtypeStruct((N, HIDDEN), jnp.float32),
    )(aggp, y0, degp, b0r)

    adj_logits, mx = pl.pallas_call(
        _logits_body,
        grid=(NB, NB),
        in_specs=[
            _vspec((BLK, HIDDEN), lambda i, j: (i, 0)),
            _vspec((BLK, HIDDEN), lambda i, j: (j, 0)),
        ],
        out_specs=[
            _vspec((BLK, BLK), lambda i, j: (i, j)),
            _vspec((1, 1), lambda i, j: (0, 0)),
        ],
        out_shape=[
            jax.ShapeDtypeStruct((N, N), jnp.float32),
            jax.ShapeDtypeStruct((1, 1), jnp.float32),
        ],
    )(h, h)

    adjr = jax.new_ref(jnp.zeros((N * N,), jnp.float32))
    sc_orig(flat2d, ones1d, adjr)
    adj_orig = adjr[...].reshape(N, N)

    adj_sampled, adj16, rowsum = pl.pallas_call(
        _sample_body,
        grid=(NB, NB),
        in_specs=[
            _vspec((BLK, HIDDEN), lambda i, j: (jnp.minimum(i, j), 0)),
            _vspec((BLK, HIDDEN), lambda i, j: (jnp.maximum(i, j), 0)),
            _vspec((BLK, BLK),
                   lambda i, j: (jnp.minimum(i, j), jnp.maximum(i, j))),
            _vspec((BLK, BLK),
                   lambda i, j: (jnp.minimum(i, j), jnp.maximum(i, j))),
            _SMEM_SPEC,
        ],
        out_specs=[
            _vspec((BLK, BLK), lambda i, j: (i, j)),
            _vspec((BLK, BLK), lambda i, j: (i, j)),
            _vspec((BLK, 1), lambda i, j: (i, 0)),
        ],
        out_shape=[
            jax.ShapeDtypeStruct((N, N), jnp.float32),
            jax.ShapeDtypeStruct((N, N), jnp.bfloat16),
            jax.ShapeDtypeStruct((N, 1), jnp.float32),
        ],
    )(h, h, adj_orig, u, mx)

    h1, z, sm, sq = pl.pallas_call(
        _h1_body,
        grid=(NB, NB),
        in_specs=[
            _vspec((BLK, BLK), lambda i, k: (i, k)),
            _vspec((BLK, HIDDEN), lambda i, k: (k, 0)),
            _vspec((BLK, 1), lambda i, k: (k, 0)),
            _vspec((BLK, HIDDEN), lambda i, k: (i, 0)),
            _vspec((BLK, 1), lambda i, k: (i, 0)),
            _vspec((1, HIDDEN), lambda i, k: (0, 0)),
            _SMEM_SPEC,
        ],
        out_specs=[
            _vspec((BLK, HIDDEN), lambda i, k: (i, 0)),
            _vspec((BLK, HIDDEN), lambda i, k: (i, 0)),
            _vspec((1, HIDDEN), lambda i, k: (0, 0)),
            _vspec((1, HIDDEN), lambda i, k: (0, 0)),
        ],
        out_shape=[
            jax.ShapeDtypeStruct((N, HIDDEN), jnp.float32),
            jax.ShapeDtypeStruct((N, HIDDEN), jnp.float32),
            jax.ShapeDtypeStruct((1, HIDDEN), jnp.float32),
            jax.ShapeDtypeStruct((1, HIDDEN), jnp.float32),
        ],
        scratch_shapes=[pltpu.VMEM((BLK, HIDDEN), jnp.float32)],
    )(adj16, xw0, rowsum, xw0, rowsum, b0r, pa)

    out = pl.pallas_call(
        _out_body,
        grid=(NB, NB),
        in_specs=[
            _vspec((BLK, BLK), lambda i, k: (i, k)),
            _vspec((BLK, HIDDEN), lambda i, k: (k, 0)),
            _vspec((BLK, 1), lambda i, k: (k, 0)),
            _vspec((BLK, HIDDEN), lambda i, k: (i, 0)),
            _vspec((BLK, 1), lambda i, k: (i, 0)),
            _vspec((1, HIDDEN), lambda i, k: (0, 0)),
            _vspec((1, HIDDEN), lambda i, k: (0, 0)),
            _vspec((1, HIDDEN), lambda i, k: (0, 0)),
            _vspec((1, HIDDEN), lambda i, k: (0, 0)),
            _vspec((HIDDEN, OUT_CH), lambda i, k: (0, 0)),
            _vspec((1, OUT_CH), lambda i, k: (0, 0)),
        ],
        out_specs=_vspec((BLK, OUT_CH), lambda i, k: (i, 0)),
        out_shape=jax.ShapeDtypeStruct((N, OUT_CH), jnp.float32),
        scratch_shapes=[pltpu.VMEM((BLK, HIDDEN), jnp.float32),
                        pltpu.VMEM((BLK, HIDDEN), jnp.float32)],
    )(adj16, h1, rowsum, h1, rowsum, sm, sq, gam, bet, W1, b1r)

    return (out, z, adj_sampled, adj_logits, adj_orig)


# speedup vs baseline: 1.6635x; 1.0542x over previous
import functools

import jax
import jax.numpy as jnp
from jax import lax
from jax.experimental import pallas as pl
from jax.experimental.pallas import tpu as pltpu
from jax.experimental.pallas import tpu_sc as plsc

N = 4096
E = 65536
IN_CH = 256
HIDDEN = 64
OUT_CH = 256
BLK = 512
NB = N // BLK

_NTILES = 32
_EPT = E // _NTILES
_ROWS_PT = _EPT // 128


@functools.lru_cache(maxsize=None)
def _sc_kernels():
    mesh = plsc.VectorSubcoreMesh(core_axis_name="c", subcore_axis_name="s")

    @functools.partial(
        pl.kernel, mesh=mesh,
        out_type=jax.ShapeDtypeStruct((2, N, 16), jnp.float32),
        compiler_params=pltpu.CompilerParams(use_tc_tiling_on_sc=False),
        scratch_types=[
            pltpu.VMEM((_ROWS_PT, 128), jnp.int32),
            pltpu.VMEM((128, 16), jnp.float32),
            pltpu.VMEM_SHARED((N, 16), jnp.float32),
        ],
    )
    def sc_deg(dst2d, zer16, ones16, out, idx_v, ones_v, hist_sh):
        c = lax.axis_index("c")
        s = lax.axis_index("s")
        wid = s * 2 + c

        @pl.when(s == 0)
        def _():
            pltpu.sync_copy(zer16, hist_sh)

        plsc.subcore_barrier()
        pltpu.sync_copy(dst2d.at[pl.ds(wid * _ROWS_PT, _ROWS_PT)], idx_v)
        pltpu.sync_copy(ones16, ones_v)
        for j in range(_ROWS_PT):
            pltpu.sync_copy(ones_v, hist_sh.at[idx_v.at[j]], add=True)
        plsc.subcore_barrier()
        pltpu.sync_copy(hist_sh.at[pl.ds(s * (N // 16), N // 16)],
                        out.at[c, pl.ds(s * (N // 16), N // 16)])

    @functools.partial(
        pl.kernel, mesh=mesh,
        out_type=jax.ShapeDtypeStruct((2, N, HIDDEN), jnp.float32),
        compiler_params=pltpu.CompilerParams(use_tc_tiling_on_sc=False),
        scratch_types=[
            pltpu.VMEM((_ROWS_PT, 128), jnp.int32),
            pltpu.VMEM((_ROWS_PT, 128), jnp.int32),
            pltpu.VMEM((128, HIDDEN), jnp.float32),
            pltpu.VMEM((128, HIDDEN), jnp.float32),
            pltpu.VMEM_SHARED((N, HIDDEN), jnp.float32),
            pltpu.SemaphoreType.DMA,
            pltpu.SemaphoreType.DMA,
        ],
    )
    def sc_agg(src2d, dst2d, y0, zer64, out,
               isrc, idst, rows_v, rows_v2, agg_sh, sem, sem2):
        c = lax.axis_index("c")
        s = lax.axis_index("s")
        wid = s * 2 + c

        @pl.when(s == 0)
        def _():
            pltpu.sync_copy(zer64, agg_sh)

        plsc.subcore_barrier()
        pltpu.sync_copy(src2d.at[pl.ds(wid * _ROWS_PT, _ROWS_PT)], isrc)
        pltpu.sync_copy(dst2d.at[pl.ds(wid * _ROWS_PT, _ROWS_PT)], idst)
        bufs = (rows_v, rows_v2)
        sems = (sem, sem2)
        descs = [None] * _ROWS_PT
        descs[0] = pltpu.async_copy(y0.at[isrc.at[0]], bufs[0], sems[0])
        for j in range(_ROWS_PT):
            descs[j].wait()
            if j + 1 < _ROWS_PT:
                descs[j + 1] = pltpu.async_copy(
                    y0.at[isrc.at[j + 1]], bufs[(j + 1) % 2], sems[(j + 1) % 2])
            pltpu.sync_copy(bufs[j % 2], agg_sh.at[idst.at[j]], add=True)
        plsc.subcore_barrier()
        pltpu.sync_copy(agg_sh.at[pl.ds(s * (N // 16), N // 16)],
                        out.at[c, pl.ds(s * (N // 16), N // 16)])

    @functools.partial(
        pl.kernel, mesh=mesh,
        out_type=jax.ShapeDtypeStruct((8,), jnp.float32),
        scratch_types=[
            pltpu.VMEM((_ROWS_PT, 128), jnp.int32),
            pltpu.VMEM((128,), jnp.float32),
        ],
    )
    def sc_orig(flat2d, ones1d, adj_ref, dummy, idx_v, ones_v):
        c = lax.axis_index("c")
        s = lax.axis_index("s")
        wid = s * 2 + c
        pltpu.sync_copy(flat2d.at[pl.ds(wid * _ROWS_PT, _ROWS_PT)], idx_v)
        pltpu.sync_copy(ones1d, ones_v)
        for j in range(_ROWS_PT):
            pltpu.sync_copy(ones_v, adj_ref.at[idx_v.at[j]])

        @pl.when(jnp.logical_and(c == 0, s == 0))
        def _():
            pltpu.sync_copy(ones_v.at[pl.ds(0, 8)], dummy)

    return sc_deg, sc_agg, sc_orig


def _prep_body(x_ref, w0_ref, degp_ref, xw_ref, y0_ref):
    xw = jnp.dot(x_ref[...].astype(jnp.bfloat16),
                 w0_ref[...].astype(jnp.bfloat16),
                 preferred_element_type=jnp.float32)
    deg = degp_ref[0, :, 0:1] + degp_ref[1, :, 0:1] + 1.0
    dis = 1.0 / jnp.sqrt(deg)
    xw_ref[...] = xw
    y0_ref[...] = (dis * xw).astype(jnp.bfloat16).astype(jnp.float32)


def _h_body(aggp_ref, y0_ref, degp_ref, b0_ref, h_ref):
    deg = degp_ref[0, :, 0:1] + degp_ref[1, :, 0:1] + 1.0
    dis = 1.0 / jnp.sqrt(deg)
    agg = aggp_ref[0] + aggp_ref[1] + y0_ref[...]
    h_ref[...] = dis * agg + b0_ref[...]


def _logits_body(ha_ref, hb_ref, out_ref, mx_ref):
    i = pl.program_id(0)
    j = pl.program_id(1)
    t = lax.dot_general(ha_ref[...].astype(jnp.bfloat16),
                        hb_ref[...].astype(jnp.bfloat16),
                        (((1,), (1,)), ((), ())),
                        preferred_element_type=jnp.float32)
    out_ref[...] = t
    prev = jnp.where((i == 0) & (j == 0),
                     jnp.full((1, 1), -jnp.inf, jnp.float32), mx_ref[...])
    mx_ref[...] = jnp.maximum(prev, jnp.max(t).reshape(1, 1))


def _sample_body(ha_ref, hb_ref, orig_ref, u_ref, m_ref,
                 samp_ref, samp16_ref, rs_ref):
    i = pl.program_id(0)
    j = pl.program_id(1)
    m = m_ref[0, 0]
    logits = lax.dot_general(ha_ref[...].astype(jnp.bfloat16),
                             hb_ref[...].astype(jnp.bfloat16),
                             (((1,), (1,)), ((), ())),
                             preferred_element_type=jnp.float32)
    ep = 0.8 * (logits / m) + 0.2 * orig_ref[...]
    ep = jnp.where(ep < 0.0, 0.0, ep)
    p = jnp.clip(ep, 1e-6, 1.0 - 1e-6)
    r = (p + u_ref[...] > 1.0).astype(jnp.float32)
    rows = lax.broadcasted_iota(jnp.int32, (BLK, BLK), 0)
    cols = lax.broadcasted_iota(jnp.int32, (BLK, BLK), 1)
    tri = jnp.where(rows < cols, r, 0.0)
    sym = tri + tri.T + jnp.where(rows == cols, 1.0, 0.0)
    out = jnp.where(i < j, r, jnp.where(i > j, r.T, sym))
    samp_ref[...] = out
    samp16_ref[...] = out.astype(jnp.bfloat16)
    prev = jnp.where(j == 0, jnp.zeros((BLK, 1), jnp.float32), rs_ref[...])
    rs_ref[...] = prev + jnp.sum(out, axis=1, keepdims=True)


def _h1_body(a_ref, xwk_ref, csk_ref, xwi_ref, csi_ref, b0_ref, pa_ref,
             h1_ref, z_ref, sm_ref, sq_ref, acc_ref):
    i = pl.program_id(0)
    k = pl.program_id(1)
    dis_k = lax.rsqrt(csk_ref[...] + 1.0)
    y1_k = xwk_ref[...] * dis_k

    @pl.when(k == 0)
    def _():
        acc_ref[...] = jnp.zeros_like(acc_ref)

    acc_ref[...] += jnp.dot(a_ref[...], y1_k.astype(jnp.bfloat16),
                            preferred_element_type=jnp.float32)

    @pl.when(k == NB - 1)
    def _():
        dis_i = lax.rsqrt(csi_ref[...] + 1.0)
        y1_i = xwi_ref[...] * dis_i
        h1 = dis_i * (acc_ref[...] + y1_i) + b0_ref[...]
        h1_ref[...] = h1
        a = pa_ref[0, 0]
        z_ref[...] = jnp.where(h1 >= 0.0, h1, a * h1)
        pv_s = jnp.where(i == 0, jnp.zeros((1, HIDDEN), jnp.float32),
                         sm_ref[...])
        pv_q = jnp.where(i == 0, jnp.zeros((1, HIDDEN), jnp.float32),
                         sq_ref[...])
        sm_ref[...] = pv_s + jnp.sum(h1, axis=0, keepdims=True)
        sq_ref[...] = pv_q + jnp.sum(h1 * h1, axis=0, keepdims=True)


def _out_body(a_ref, h1k_ref, csk_ref, h1i_ref, csi_ref, sm_ref, sq_ref,
              gam_ref, bet_ref, w1_ref, b1_ref, out_ref, acc_ref, gself_ref):
    k = pl.program_id(1)
    mean = sm_ref[...] / N
    var = sq_ref[...] / N - mean * mean
    inv = lax.rsqrt(var + 1e-5)

    def bn_g(h1, cs):
        hbn = (h1 - mean) * inv * gam_ref[...] + bet_ref[...]
        h2 = jnp.maximum(hbn, 0.0)
        return lax.rsqrt(cs + 1.0) * h2

    g_k = bn_g(h1k_ref[...], csk_ref[...])

    @pl.when(k == 0)
    def _():
        acc_ref[...] = jnp.zeros_like(acc_ref)

    acc_ref[...] += jnp.dot(a_ref[...], g_k.astype(jnp.bfloat16),
                            preferred_element_type=jnp.float32)

    @pl.when(k == NB - 1)
    def _():
        g_i = bn_g(h1i_ref[...], csi_ref[...])
        s = acc_ref[...] + g_i
        dis_i = lax.rsqrt(csi_ref[...] + 1.0)
        out_ref[...] = dis_i * jnp.dot(s, w1_ref[...],
                                       preferred_element_type=jnp.float32) \
            + b1_ref[...]


def _vspec(shape, imap):
    return pl.BlockSpec(shape, imap)


_SMEM_SPEC = pl.BlockSpec(memory_space=pltpu.SMEM)


def kernel(x, edge_index, W0, b0, W1, b1, prelu_a, gamma, beta):
    src = edge_index[0].astype(jnp.int32)
    dst = edge_index[1].astype(jnp.int32)
    src2d = src.reshape(E // 128, 128)
    dst2d = dst.reshape(E // 128, 128)
    flat2d = (src * N + dst).reshape(E // 128, 128)
    u = jax.random.uniform(jax.random.key(42), (N, N),
                           minval=1e-6, maxval=1.0 - 1e-6)
    zer16 = jnp.zeros((N, 16), jnp.float32)
    ones16 = jnp.ones((128, 16), jnp.float32)
    zer64 = jnp.zeros((N, HIDDEN), jnp.float32)
    ones1d = jnp.ones((128,), jnp.float32)
    b0r = b0.reshape(1, HIDDEN)
    b1r = b1.reshape(1, OUT_CH)
    gam = gamma.reshape(1, HIDDEN)
    bet = beta.reshape(1, HIDDEN)
    pa = prelu_a.reshape(1, 1)

    sc_deg, sc_agg, sc_orig = _sc_kernels()

    degp = sc_deg(dst2d, zer16, ones16)

    xw0, y0 = pl.pallas_call(
        _prep_body,
        grid=(NB,),
        in_specs=[
            _vspec((BLK, IN_CH), lambda i: (i, 0)),
            _vspec((IN_CH, HIDDEN), lambda i: (0, 0)),
            _vspec((2, BLK, 16), lambda i: (0, i, 0)),
        ],
        out_specs=[
            _vspec((BLK, HIDDEN), lambda i: (i, 0)),
            _vspec((BLK, HIDDEN), lambda i: (i, 0)),
        ],
        out_shape=[
            jax.ShapeDtypeStruct((N, HIDDEN), jnp.float32),
            jax.ShapeDtypeStruct((N, HIDDEN), jnp.float32),
        ],
    )(x, W0, degp)

    aggp = sc_agg(src2d, dst2d, y0, zer64)

    h = pl.pallas_call(
        _h_body,
        grid=(NB,),
        in_specs=[
            _vspec((2, BLK, HIDDEN), lambda i: (0, i, 0)),
            _vspec((BLK, HIDDEN), lambda i: (i, 0)),
            _vspec((2, BLK, 16), lambda i: (0, i, 0)),
            _vspec((1, HIDDEN), lambda i: (0, 0)),
        ],
        out_specs=_vspec((BLK, HIDDEN), lambda i: (i, 0)),
        out_shape=jax.ShapeDtypeStruct((N, HIDDEN), jnp.float32),
    )(aggp, y0, degp, b0r)

    adj_logits, mx = pl.pallas_call(
        _logits_body,
        grid=(NB, NB),
        in_specs=[
            _vspec((BLK, HIDDEN), lambda i, j: (i, 0)),
            _vspec((BLK, HIDDEN), lambda i, j: (j, 0)),
        ],
        out_specs=[
            _vspec((BLK, BLK), lambda i, j: (i, j)),
            _vspec((1, 1), lambda i, j: (0, 0)),
        ],
        out_shape=[
            jax.ShapeDtypeStruct((N, N), jnp.float32),
            jax.ShapeDtypeStruct((1, 1), jnp.float32),
        ],
    )(h, h)

    adjr = jax.new_ref(jnp.zeros((N * N,), jnp.float32))
    sc_orig(flat2d, ones1d, adjr)
    adj_orig = adjr[...].reshape(N, N)

    adj_sampled, adj16, rowsum = pl.pallas_call(
        _sample_body,
        grid=(NB, NB),
        in_specs=[
            _vspec((BLK, HIDDEN), lambda i, j: (jnp.minimum(i, j), 0)),
            _vspec((BLK, HIDDEN), lambda i, j: (jnp.maximum(i, j), 0)),
            _vspec((BLK, BLK),
                   lambda i, j: (jnp.minimum(i, j), jnp.maximum(i, j))),
            _vspec((BLK, BLK),
                   lambda i, j: (jnp.minimum(i, j), jnp.maximum(i, j))),
            _SMEM_SPEC,
        ],
        out_specs=[
            _vspec((BLK, BLK), lambda i, j: (i, j)),
            _vspec((BLK, BLK), lambda i, j: (i, j)),
            _vspec((BLK, 1), lambda i, j: (i, 0)),
        ],
        out_shape=[
            jax.ShapeDtypeStruct((N, N), jnp.float32),
            jax.ShapeDtypeStruct((N, N), jnp.bfloat16),
            jax.ShapeDtypeStruct((N, 1), jnp.float32),
        ],
    )(h, h, adj_orig, u, mx)

    h1, z, sm, sq = pl.pallas_call(
        _h1_body,
        grid=(NB, NB),
        in_specs=[
            _vspec((BLK, BLK), lambda i, k: (i, k)),
            _vspec((BLK, HIDDEN), lambda i, k: (k, 0)),
            _vspec((BLK, 1), lambda i, k: (k, 0)),
            _vspec((BLK, HIDDEN), lambda i, k: (i, 0)),
            _vspec((BLK, 1), lambda i, k: (i, 0)),
            _vspec((1, HIDDEN), lambda i, k: (0, 0)),
            _SMEM_SPEC,
        ],
        out_specs=[
            _vspec((BLK, HIDDEN), lambda i, k: (i, 0)),
            _vspec((BLK, HIDDEN), lambda i, k: (i, 0)),
            _vspec((1, HIDDEN), lambda i, k: (0, 0)),
            _vspec((1, HIDDEN), lambda i, k: (0, 0)),
        ],
        out_shape=[
            jax.ShapeDtypeStruct((N, HIDDEN), jnp.float32),
            jax.ShapeDtypeStruct((N, HIDDEN), jnp.float32),
            jax.ShapeDtypeStruct((1, HIDDEN), jnp.float32),
            jax.ShapeDtypeStruct((1, HIDDEN), jnp.float32),
        ],
        scratch_shapes=[pltpu.VMEM((BLK, HIDDEN), jnp.float32)],
    )(adj16, xw0, rowsum, xw0, rowsum, b0r, pa)

    out = pl.pallas_call(
        _out_body,
        grid=(NB, NB),
        in_specs=[
            _vspec((BLK, BLK), lambda i, k: (i, k)),
            _vspec((BLK, HIDDEN), lambda i, k: (k, 0)),
            _vspec((BLK, 1), lambda i, k: (k, 0)),
            _vspec((BLK, HIDDEN), lambda i, k: (i, 0)),
            _vspec((BLK, 1), lambda i, k: (i, 0)),
            _vspec((1, HIDDEN), lambda i, k: (0, 0)),
            _vspec((1, HIDDEN), lambda i, k: (0, 0)),
            _vspec((1, HIDDEN), lambda i, k: (0, 0)),
            _vspec((1, HIDDEN), lambda i, k: (0, 0)),
            _vspec((HIDDEN, OUT_CH), lambda i, k: (0, 0)),
            _vspec((1, OUT_CH), lambda i, k: (0, 0)),
        ],
        out_specs=_vspec((BLK, OUT_CH), lambda i, k: (i, 0)),
        out_shape=jax.ShapeDtypeStruct((N, OUT_CH), jnp.float32),
        scratch_shapes=[pltpu.VMEM((BLK, HIDDEN), jnp.float32),
                        pltpu.VMEM((BLK, HIDDEN), jnp.float32)],
    )(adj16, h1, rowsum, h1, rowsum, sm, sq, gam, bet, W1, b1r)

    return (out, z, adj_sampled, adj_logits, adj_orig)
